# baseline (speedup 1.0000x reference)
import functools

import jax
import jax.numpy as jnp
from jax import lax
from jax.experimental import pallas as pl
from jax.experimental.pallas import tpu as pltpu
from jax.experimental.pallas import tpu_sc as plsc

_NC = 2
_NS = 16
_NW = _NC * _NS


def _sc_gather_mean(idx, table):
    B, L = idx.shape
    _, D = table.shape
    bpw = B // _NW

    mesh = plsc.VectorSubcoreMesh(core_axis_name="c", subcore_axis_name="s")

    @functools.partial(
        pl.kernel,
        out_type=jax.ShapeDtypeStruct((B, D), jnp.float32),
        mesh=mesh,
        scratch_types=[
            pltpu.VMEM((bpw, L), jnp.int32),
            pltpu.VMEM((bpw, L, D), jnp.float32),
            pltpu.VMEM((bpw, D), jnp.float32),
            pltpu.SemaphoreType.DMA,
        ],
        compiler_params=pltpu.CompilerParams(use_tc_tiling_on_sc=False),
    )
    def gather_mean(idx_hbm, table_hbm, out_hbm, idx_v, rows_v, ctx_v, sem):
        wid = lax.axis_index("s") * _NC + lax.axis_index("c")
        base = wid * bpw
        pltpu.sync_copy(idx_hbm.at[pl.ds(base, bpw)], idx_v)
        copies = [
            pltpu.async_copy(table_hbm.at[idx_v.at[b]], rows_v.at[b], sem)
            for b in range(bpw)
        ]
        for c in copies:
            c.wait()

        scale = jnp.float32(1.0 / L)

        def pool_row(b, carry):
            for h in range(D // 16):
                acc = rows_v[b, 0, pl.ds(h * 16, 16)]
                for j in range(1, L):
                    acc = acc + rows_v[b, j, pl.ds(h * 16, 16)]
                ctx_v[b, pl.ds(h * 16, 16)] = acc * scale
            return carry

        lax.fori_loop(0, bpw, pool_row, 0)
        pltpu.sync_copy(ctx_v, out_hbm.at[pl.ds(base, bpw)])

    return gather_mean(idx, table)


def _sc_gather_mean_t(idx, table):
    B, L = idx.shape
    V, D = table.shape
    assert D == _NW
    idx_t = idx.T.reshape(L, B // 16, 16)
    tab_t = table.T
    G = B // 16
    gc = 32

    mesh = plsc.VectorSubcoreMesh(core_axis_name="c", subcore_axis_name="s")

    @functools.partial(
        pl.kernel,
        out_type=jax.ShapeDtypeStruct((D, G, 16), jnp.float32),
        mesh=mesh,
        scratch_types=[
            pltpu.VMEM((V,), jnp.float32),
            pltpu.VMEM((L, gc, 16), jnp.int32),
            pltpu.VMEM((G, 16), jnp.float32),
            pltpu.SemaphoreType.DMA,
        ],
        compiler_params=pltpu.CompilerParams(
            use_tc_tiling_on_sc=False, needs_layout_passes=False
        ),
    )
    def gather_mean(idx_hbm, tab_hbm, out_hbm, row_v, idx_v, acc_v, sem):
        d = lax.axis_index("s") * _NC + lax.axis_index("c")
        pltpu.sync_copy(tab_hbm.at[d], row_v)
        scale = jnp.float32(1.0 / L)

        def make_pool_group(cbase):
            def pool_group(g, carry):
                acc = plsc.load_gather(row_v, [idx_v[0, g, :]])
                for j in range(1, L):
                    acc = acc + plsc.load_gather(row_v, [idx_v[j, g, :]])
                acc_v[cbase + g, :] = acc * scale
                return carry
            return pool_group

        for c in range(G // gc):
            pltpu.sync_copy(idx_hbm.at[:, pl.ds(c * gc, gc)], idx_v)
            lax.fori_loop(0, gc, make_pool_group(c * gc), 0)
        pltpu.sync_copy(acc_v, out_hbm.at[d])

    return gather_mean(idx_t, tab_t).reshape(D, B)


def _tc_dense_t(ctx, W, b, bv):
    D, B = ctx.shape
    V = W.shape[1]
    ctx_t = ctx

    nsteps = pl.cdiv(V, bv)
    tail = V - (nsteps - 1) * bv
    nbuf = 3

    def mm(w_ref, ctx_ref, b_ref, out_ref, acc, sems):
        i = pl.program_id(0)
        s = lax.rem(i, nbuf)

        @pl.when(i >= nbuf)
        def _wait_prior():
            pltpu.make_async_copy(
                acc.at[s], out_ref.at[pl.ds((i - nbuf) * bv, bv)], sems.at[s]
            ).wait()

        bias = jnp.reshape(b_ref[...], (bv, 1))
        acc[s] = (
            lax.dot_general(
                w_ref[...],
                ctx_ref[...],
                dimension_numbers=(((0,), (0,)), ((), ())),
                preferred_element_type=jnp.float32,
            )
            + bias
        )

        @pl.when(i < nsteps - 1)
        def _start_full():
            pltpu.make_async_copy(
                acc.at[s], out_ref.at[pl.ds(i * bv, bv)], sems.at[s]
            ).start()

        @pl.when(i == nsteps - 1)
        def _finish():
            pltpu.make_async_copy(
                acc.at[s, pl.ds(0, tail)],
                out_ref.at[pl.ds(i * bv, tail)],
                sems.at[s],
            ).start()
            for step in range(nsteps - nbuf, nsteps):
                size = bv if step < nsteps - 1 else tail
                sj = step % nbuf
                pltpu.make_async_copy(
                    acc.at[sj, pl.ds(0, size)],
                    out_ref.at[pl.ds(step * bv, size)],
                    sems.at[sj],
                ).wait()

    out_t = pl.pallas_call(
        mm,
        grid=(nsteps,),
        in_specs=[
            pl.BlockSpec((D, bv), lambda i: (0, i)),
            pl.BlockSpec((D, B), lambda i: (0, 0)),
            pl.BlockSpec((bv,), lambda i: (i,)),
        ],
        out_specs=pl.BlockSpec(memory_space=pl.ANY),
        out_shape=jax.ShapeDtypeStruct((V, B), jnp.float32),
        scratch_shapes=[
            pltpu.VMEM((nbuf, bv, B), jnp.float32),
            pltpu.SemaphoreType.DMA((nbuf,)),
        ],
        compiler_params=pltpu.CompilerParams(
            dimension_semantics=("arbitrary",),
            vmem_limit_bytes=100 * 1024 * 1024,
        ),
    )(W, ctx_t, b)
    return out_t.T


def kernel(inputs, table, W, b):
    ctx_t = _sc_gather_mean_t(inputs.astype(jnp.int32), table)
    return _tc_dense_t(ctx_t, W, b, 4096)


# speedup vs baseline: 1.0053x; 1.0053x over previous
import functools

import jax
import jax.numpy as jnp
from jax import lax
from jax.experimental import pallas as pl
from jax.experimental.pallas import tpu as pltpu
from jax.experimental.pallas import tpu_sc as plsc

_NC = 2
_NS = 16
_NW = _NC * _NS


def _sc_gather_mean(idx, table):
    B, L = idx.shape
    _, D = table.shape
    bpw = B // _NW

    mesh = plsc.VectorSubcoreMesh(core_axis_name="c", subcore_axis_name="s")

    @functools.partial(
        pl.kernel,
        out_type=jax.ShapeDtypeStruct((B, D), jnp.float32),
        mesh=mesh,
        scratch_types=[
            pltpu.VMEM((bpw, L), jnp.int32),
            pltpu.VMEM((bpw, L, D), jnp.float32),
            pltpu.VMEM((bpw, D), jnp.float32),
            pltpu.SemaphoreType.DMA,
        ],
        compiler_params=pltpu.CompilerParams(use_tc_tiling_on_sc=False),
    )
    def gather_mean(idx_hbm, table_hbm, out_hbm, idx_v, rows_v, ctx_v, sem):
        wid = lax.axis_index("s") * _NC + lax.axis_index("c")
        base = wid * bpw
        pltpu.sync_copy(idx_hbm.at[pl.ds(base, bpw)], idx_v)
        copies = [
            pltpu.async_copy(table_hbm.at[idx_v.at[b]], rows_v.at[b], sem)
            for b in range(bpw)
        ]
        for c in copies:
            c.wait()

        scale = jnp.float32(1.0 / L)

        def pool_row(b, carry):
            for h in range(D // 16):
                acc = rows_v[b, 0, pl.ds(h * 16, 16)]
                for j in range(1, L):
                    acc = acc + rows_v[b, j, pl.ds(h * 16, 16)]
                ctx_v[b, pl.ds(h * 16, 16)] = acc * scale
            return carry

        lax.fori_loop(0, bpw, pool_row, 0)
        pltpu.sync_copy(ctx_v, out_hbm.at[pl.ds(base, bpw)])

    return gather_mean(idx, table)


def _sc_gather_mean_t(idx, table):
    B, L = idx.shape
    V, D = table.shape
    assert D == _NW
    idx_t = idx.T.reshape(L, B // 16, 16)
    tab_t = table.T
    G = B // 16
    gc = 16
    nchunk = G // gc

    mesh = plsc.VectorSubcoreMesh(core_axis_name="c", subcore_axis_name="s")

    @functools.partial(
        pl.kernel,
        out_type=jax.ShapeDtypeStruct((D, G, 16), jnp.float32),
        mesh=mesh,
        scratch_types=[
            pltpu.VMEM((V,), jnp.float32),
            pltpu.VMEM((2, L, gc, 16), jnp.int32),
            pltpu.VMEM((G, 16), jnp.float32),
            pltpu.SemaphoreType.DMA,
            pltpu.SemaphoreType.DMA,
            pltpu.SemaphoreType.DMA,
        ],
        compiler_params=pltpu.CompilerParams(
            use_tc_tiling_on_sc=False, needs_layout_passes=False
        ),
    )
    def gather_mean(idx_hbm, tab_hbm, out_hbm, row_v, idx_v, acc_v,
                    row_sem, sem0, sem1):
        d = lax.axis_index("s") * _NC + lax.axis_index("c")
        row_cp = pltpu.async_copy(tab_hbm.at[d], row_v, row_sem)
        sems = (sem0, sem1)
        idx_cp = [
            pltpu.async_copy(
                idx_hbm.at[:, pl.ds(c * gc, gc)], idx_v.at[c % 2], sems[c % 2]
            )
            if c < 2
            else None
            for c in range(nchunk)
        ]
        row_cp.wait()
        scale = jnp.float32(1.0 / L)

        def make_pool_group(buf, cbase):
            def pool_group(g, carry):
                acc = plsc.load_gather(row_v, [idx_v[buf, 0, g, :]])
                for j in range(1, L):
                    acc = acc + plsc.load_gather(row_v, [idx_v[buf, j, g, :]])
                acc_v[cbase + g, :] = acc * scale
                return carry
            return pool_group

        for c in range(nchunk):
            idx_cp[c].wait()
            lax.fori_loop(0, gc, make_pool_group(c % 2, c * gc), 0)
            if c + 2 < nchunk:
                idx_cp[c + 2] = pltpu.async_copy(
                    idx_hbm.at[:, pl.ds((c + 2) * gc, gc)],
                    idx_v.at[c % 2],
                    sems[c % 2],
                )
        pltpu.sync_copy(acc_v, out_hbm.at[d])

    return gather_mean(idx_t, tab_t).reshape(D, B)


def _tc_dense_t(ctx, W, b, bv):
    D, B = ctx.shape
    V = W.shape[1]
    ctx_t = ctx

    nsteps = pl.cdiv(V, bv)
    tail = V - (nsteps - 1) * bv
    nbuf = 6

    def mm(w_ref, ctx_ref, b_ref, out_ref, acc, sems):
        i = pl.program_id(0)
        s = lax.rem(i, nbuf)

        @pl.when(i >= nbuf)
        def _wait_prior():
            pltpu.make_async_copy(
                acc.at[s], out_ref.at[pl.ds((i - nbuf) * bv, bv)], sems.at[s]
            ).wait()

        bias = jnp.reshape(b_ref[...], (bv, 1))
        acc[s] = (
            lax.dot_general(
                w_ref[...],
                ctx_ref[...],
                dimension_numbers=(((0,), (0,)), ((), ())),
                preferred_element_type=jnp.float32,
            )
            + bias
        )

        @pl.when(i < nsteps - 1)
        def _start_full():
            pltpu.make_async_copy(
                acc.at[s], out_ref.at[pl.ds(i * bv, bv)], sems.at[s]
            ).start()

        @pl.when(i == nsteps - 1)
        def _finish():
            pltpu.make_async_copy(
                acc.at[s, pl.ds(0, tail)],
                out_ref.at[pl.ds(i * bv, tail)],
                sems.at[s],
            ).start()
            for step in range(nsteps - nbuf, nsteps):
                size = bv if step < nsteps - 1 else tail
                sj = step % nbuf
                pltpu.make_async_copy(
                    acc.at[sj, pl.ds(0, size)],
                    out_ref.at[pl.ds(step * bv, size)],
                    sems.at[sj],
                ).wait()

    out_t = pl.pallas_call(
        mm,
        grid=(nsteps,),
        in_specs=[
            pl.BlockSpec((D, bv), lambda i: (0, i)),
            pl.BlockSpec((D, B), lambda i: (0, 0)),
            pl.BlockSpec((bv,), lambda i: (i,)),
        ],
        out_specs=pl.BlockSpec(memory_space=pl.ANY),
        out_shape=jax.ShapeDtypeStruct((V, B), jnp.float32),
        scratch_shapes=[
            pltpu.VMEM((nbuf, bv, B), jnp.float32),
            pltpu.SemaphoreType.DMA((nbuf,)),
        ],
        compiler_params=pltpu.CompilerParams(
            dimension_semantics=("arbitrary",),
            vmem_limit_bytes=100 * 1024 * 1024,
        ),
    )(W, ctx_t, b)
    return out_t.T


def kernel(inputs, table, W, b):
    ctx_t = _sc_gather_mean_t(inputs.astype(jnp.int32), table)
    return _tc_dense_t(ctx_t, W, b, 2048)
